# baseline (speedup 1.0000x reference)
import functools

import jax
import jax.numpy as jnp
from jax import lax
from jax.experimental import pallas as pl
from jax.experimental.pallas import tpu as pltpu
from jax.experimental.pallas import tpu_sc as plsc

N = 10000
E = 320000
P = 100000
R = 8
D = 128
NC = 2
NS = 16
L = 16
NW = NC * NS
BINS = 81920
NB = 64
BROWS = 160
NP = NB * BROWS
EPT = E // NW
RECB = EPT + 8 * NB

_MESH = dict(core_axis_name="c", subcore_axis_name="s")
_SC_PARAMS = pltpu.CompilerParams(needs_layout_passes=False)


def _count():
    K = 80

    @functools.partial(
        pl.kernel,
        out_type=jax.ShapeDtypeStruct((NW * BINS,), jnp.float32),
        mesh=plsc.VectorSubcoreMesh(**_MESH),
        compiler_params=_SC_PARAMS,
        scratch_types=[
            pltpu.VMEM((BINS,), jnp.float32),
            pltpu.VMEM((K,), jnp.int32),
            pltpu.VMEM((K,), jnp.int32),
        ],
    )
    def body(dst_hbm, typ_hbm, hist_hbm, cnt, dstb, typb):
        c = lax.axis_index("c")
        s = lax.axis_index("s")
        wid = s * NC + c
        zeros16 = lax.broadcast(jnp.float32(0), (L,))
        ones16 = lax.broadcast(jnp.float32(1), (L,))

        def zero_body(i, carry):
            cnt[pl.ds(i * L, L)] = zeros16
            return carry
        lax.fori_loop(0, BINS // L, zero_body, 0)

        def p1(i, carry):
            eb = wid * EPT + i * K
            pltpu.sync_copy(dst_hbm.at[pl.ds(eb, K)], dstb)
            pltpu.sync_copy(typ_hbm.at[pl.ds(eb, K)], typb)

            def inner(v, carry2):
                idx = dstb[pl.ds(v * L, L)] * R + typb[pl.ds(v * L, L)]
                plsc.addupdate_scatter(cnt, [idx], ones16)
                return carry2
            lax.fori_loop(0, K // L, inner, 0)
            return carry
        lax.fori_loop(0, EPT // K, p1, 0)

        pltpu.sync_copy(cnt, hist_hbm.at[pl.ds(wid * BINS, BINS)])

    return body


def _tc_reduce_hist(hist):
    BB = 8

    def kern(h_ref, o_ref):
        o_ref[...] = jnp.sum(h_ref[...], axis=0)

    out = pl.pallas_call(
        kern,
        grid=(BINS // (BB * 128),),
        in_specs=[pl.BlockSpec((NW, BB, 128), lambda i: (0, i, 0))],
        out_specs=pl.BlockSpec((BB, 128), lambda i: (i, 0)),
        out_shape=jax.ShapeDtypeStruct((BINS // 128, 128), jnp.float32),
    )(hist.reshape(NW, BINS // 128, 128))
    return out.reshape(BINS)


def _bucketize():
    K = 80

    @functools.partial(
        pl.kernel,
        out_type=[
            jax.ShapeDtypeStruct((NW * RECB + K,), jnp.int32),
            jax.ShapeDtypeStruct((NW * RECB + K,), jnp.int32),
            jax.ShapeDtypeStruct((NW * RECB + K,), jnp.int32),
            jax.ShapeDtypeStruct((NW * NB,), jnp.int32),
        ],
        mesh=plsc.VectorSubcoreMesh(**_MESH),
        compiler_params=_SC_PARAMS,
        scratch_types=[
            pltpu.VMEM((BINS,), jnp.float32),
            pltpu.VMEM((RECB,), jnp.int32),
            pltpu.VMEM((RECB,), jnp.int32),
            pltpu.VMEM((RECB,), jnp.int32),
            pltpu.VMEM((NB + L,), jnp.int32),
            pltpu.VMEM((NB,), jnp.float32),
            pltpu.VMEM((K,), jnp.int32),
            pltpu.VMEM((K,), jnp.int32),
            pltpu.VMEM((K,), jnp.int32),
            pltpu.VMEM((K,), jnp.int32),
            pltpu.VMEM((K,), jnp.int32),
            pltpu.VMEM((K,), jnp.int32),
            pltpu.VMEM((K,), jnp.int32),
            pltpu.SMEM((NB,), jnp.int32),
        ],
    )
    def body(src_hbm, typ_hbm, dst_hbm, cnt_hbm,
             recg_hbm, recl_hbm, reci_hbm, cnts_hbm,
             cntb, gblk, lblk, iblk, bcnt, bcntf, srcb, typb, dstb, bktb,
             gidxb, lrowb, ivib, offs):
        c = lax.axis_index("c")
        s = lax.axis_index("s")
        wid = s * NC + c
        zeros16 = lax.broadcast(jnp.float32(0), (L,))
        ones16 = lax.broadcast(jnp.float32(1), (L,))

        pltpu.sync_copy(cnt_hbm, cntb)
        for v in range(NB // L):
            bcntf[pl.ds(v * L, L)] = zeros16

        def p1(i, carry):
            eb = wid * EPT + i * K
            pltpu.sync_copy(dst_hbm.at[pl.ds(eb, K)], dstb)

            def inner(v, carry2):
                bv = dstb[pl.ds(v * L, L)] // BROWS
                plsc.addupdate_scatter(bcntf, [bv], ones16)
                return carry2
            lax.fori_loop(0, K // L, inner, 0)
            return carry
        lax.fori_loop(0, EPT // K, p1, 0)

        for v in range(NB // L):
            bcnt[pl.ds(v * L, L)] = bcntf[pl.ds(v * L, L)].astype(jnp.int32)
        pltpu.sync_copy(bcnt.at[pl.ds(0, NB)],
                        cnts_hbm.at[pl.ds(wid * NB, NB)])

        def cum(b, off):
            offs[b] = off
            n = plsc.load_gather(bcnt, [lax.broadcast(b, (L,))])[0]
            return off + ((n + 7) // 8) * 8
        lax.fori_loop(0, NB, cum, 0)

        def p3(i, carry):
            eb = wid * EPT + i * K
            pltpu.sync_copy(src_hbm.at[pl.ds(eb, K)], srcb)
            pltpu.sync_copy(typ_hbm.at[pl.ds(eb, K)], typb)
            pltpu.sync_copy(dst_hbm.at[pl.ds(eb, K)], dstb)

            def vect(v, carry2):
                sl = pl.ds(v * L, L)
                dv = dstb[sl]
                tv = typb[sl]
                bv = dv // BROWS
                bktb[sl] = bv
                lrowb[sl] = dv - bv * BROWS
                gidxb[sl] = tv * N + srcb[sl]
                cv = plsc.load_gather(cntb, [dv * R + tv])
                ivib[sl] = plsc.bitcast(1.0 / jnp.maximum(cv, 1.0), jnp.int32)
                return carry2
            lax.fori_loop(0, K // L, vect, 0)

            def place(b, carry2):
                def pv(v, off):
                    sl = pl.ds(v * L, L)
                    m = bktb[sl] == b
                    mi = m.astype(jnp.int32)
                    nm = jnp.sum(mi)

                    @pl.when(nm > 0)
                    def _():
                        pos = off + plsc.cumsum(mi) - mi
                        plsc.store_scatter(gblk, [pos], gidxb[sl], mask=m)
                        plsc.store_scatter(lblk, [pos], lrowb[sl], mask=m)
                        plsc.store_scatter(iblk, [pos], ivib[sl], mask=m)
                    return off + nm
                offs[b] = lax.fori_loop(0, K // L, pv, offs[b])
                return carry2
            lax.fori_loop(0, NB, place, 0)
            return carry
        lax.fori_loop(0, EPT // K, p3, 0)

        pltpu.sync_copy(gblk, recg_hbm.at[pl.ds(wid * RECB, RECB)])
        pltpu.sync_copy(lblk, recl_hbm.at[pl.ds(wid * RECB, RECB)])
        pltpu.sync_copy(iblk, reci_hbm.at[pl.ds(wid * RECB, RECB)])

    return body


def _aggregate():
    K = 128

    @functools.partial(
        pl.kernel,
        out_type=jax.ShapeDtypeStruct((NP, D), jnp.float32),
        mesh=plsc.VectorSubcoreMesh(**_MESH),
        compiler_params=_SC_PARAMS,
        scratch_types=[
            pltpu.VMEM_SHARED((NS * BROWS, D), jnp.float32),
            pltpu.VMEM((NW * NB + L,), jnp.int32),
            pltpu.VMEM((K,), jnp.int32),
            pltpu.VMEM((K,), jnp.int32),
            pltpu.VMEM((K,), jnp.int32),
            pltpu.VMEM((K,), jnp.int32),
            pltpu.VMEM((K,), jnp.float32),
            pltpu.VMEM((K, D), jnp.float32),
            pltpu.VMEM((K,), jnp.int32),
            pltpu.VMEM((K,), jnp.int32),
            pltpu.VMEM((K,), jnp.int32),
            pltpu.VMEM((K,), jnp.int32),
            pltpu.VMEM((K,), jnp.float32),
            pltpu.VMEM((K, D), jnp.float32),
            pltpu.SMEM((NW,), jnp.int32),
            pltpu.SemaphoreType.DMA,
            pltpu.SemaphoreType.DMA,
            pltpu.SemaphoreType.DMA,
            pltpu.SemaphoreType.DMA,
        ],
    )
    def body(z_hbm, recg_hbm, recl_hbm, reci_hbm, cnts_hbm, out_hbm,
             acc, cntsb, gidx0, lrow0, sidx0, ivi0, iv0, rows0,
             gidx1, lrow1, sidx1, ivi1, iv1, rows1,
             offsB, rsem, gsem, ssem0, ssem1):
        c = lax.axis_index("c")
        s = lax.axis_index("s")
        wid = s * NC + c
        row0 = s * BROWS
        zeros16 = lax.broadcast(jnp.float32(0), (L,))
        iota = lax.iota(jnp.int32, L)
        bufs = ((gidx0, lrow0, sidx0, ivi0, iv0, rows0, ssem0),
                (gidx1, lrow1, sidx1, ivi1, iv1, rows1, ssem1))

        pltpu.sync_copy(cnts_hbm, cntsb.at[pl.ds(0, NW * NB)])

        def rec_copies(base, i, bset):
            g, lr, _, ivi, _, _, _ = bset
            return (
                pltpu.make_async_copy(
                    recg_hbm.at[pl.ds(base + i * K, K)], g, rsem),
                pltpu.make_async_copy(
                    recl_hbm.at[pl.ds(base + i * K, K)], lr, rsem),
                pltpu.make_async_copy(
                    reci_hbm.at[pl.ds(base + i * K, K)], ivi, rsem),
            )

        def fire_rec(base, i, bset):
            for cp in rec_copies(base, i, bset):
                cp.start()

        def drain_rec(base, i, bset):
            for cp in rec_copies(base, i, bset):
                cp.wait()

        def vect(bset, nval):
            g, lr, sx, ivi, iv, _, _ = bset

            def v_(v, carry):
                sl = pl.ds(v * L, L)
                valid = (v * L + iota) < nval
                g[sl] = jnp.where(valid, g[sl], 0)
                sx[sl] = jnp.where(valid, lr[sl] + row0, row0)
                ivf = plsc.bitcast(ivi[sl], jnp.float32)
                iv[sl] = jnp.where(valid, ivf, 0.0)
                return carry
            lax.fori_loop(0, K // L, v_, 0)

        def gather_cp(bset):
            g, _, _, _, _, rows, _ = bset
            return pltpu.make_async_copy(z_hbm.at[g], rows, gsem)

        def scat_cp(bset):
            _, _, sx, _, _, rows, sem = bset
            return pltpu.make_async_copy(rows, acc.at[sx], sem)

        def scale_and_scatter(bset):
            _, _, _, _, iv, rows, _ = bset

            def u_(j, carry):
                ivs = plsc.load_gather(iv, [lax.broadcast(j, (L,))])
                for q in range(D // L):
                    rows[j, pl.ds(q * L, L)] = rows[j, pl.ds(q * L, L)] * ivs
                return carry
            lax.fori_loop(0, K, u_, 0)
            scat_cp(bset).start(add=True)

        for p in range(2):
            b_id = p * NW + wid

            def zr(i, carry):
                for q in range(D // L):
                    rows0[i, pl.ds(q * L, L)] = zeros16
                return carry
            lax.fori_loop(0, K, zr, 0)
            pltpu.sync_copy(rows0, acc.at[pl.ds(row0, K)])
            pltpu.sync_copy(rows0.at[pl.ds(0, BROWS - K)],
                            acc.at[pl.ds(row0 + K, BROWS - K)])

            def po(t, carry):
                def po2(bb, off):
                    n = plsc.load_gather(
                        cntsb, [lax.broadcast(t * NB + bb, (L,))])[0]
                    ne = ((n + 7) // 8) * 8
                    return off + jnp.where(bb < b_id, ne, 0)
                offsB[t] = lax.fori_loop(0, NB, po2, 0)
                return carry
            lax.fori_loop(0, NW, po, 0)

            def per_src(t, carry):
                n = plsc.load_gather(
                    cntsb, [lax.broadcast(t * NB + b_id, (L,))])[0]
                base = pl.multiple_of(t * RECB + offsB[t], 8)
                nch = (n + K - 1) // K

                @pl.when(nch > 0)
                def _():
                    fire_rec(base, 0, bufs[0])
                    drain_rec(base, 0, bufs[0])
                    vect(bufs[0], n)
                    gather_cp(bufs[0]).start()

                    def pair(ii, carry2):
                        i0 = 2 * ii

                        def halfstep(i, cur, nxt):
                            @pl.when(i + 1 < nch)
                            def _():
                                fire_rec(base, i + 1, nxt)
                            gather_cp(cur).wait()

                            @pl.when(i + 1 < nch)
                            def _():
                                drain_rec(base, i + 1, nxt)
                                vect(nxt, n - (i + 1) * K)

                                @pl.when(i >= 1)
                                def _():
                                    scat_cp(nxt).wait()
                                gather_cp(nxt).start()
                            scale_and_scatter(cur)

                        halfstep(i0, bufs[0], bufs[1])

                        @pl.when(i0 + 1 < nch)
                        def _():
                            halfstep(i0 + 1, bufs[1], bufs[0])
                        return carry2
                    lax.fori_loop(0, (nch + 1) // 2, pair, 0)

                    odd_last = (nch % 2) == 0

                    @pl.when(odd_last)
                    def _():
                        scat_cp(bufs[1]).wait()

                        @pl.when(nch >= 2)
                        def _():
                            scat_cp(bufs[0]).wait()

                    @pl.when(jnp.logical_not(odd_last))
                    def _():
                        scat_cp(bufs[0]).wait()

                        @pl.when(nch >= 2)
                        def _():
                            scat_cp(bufs[1]).wait()
                return carry
            lax.fori_loop(0, NW, per_src, 0)

            pltpu.sync_copy(acc.at[pl.ds(row0, BROWS)],
                            out_hbm.at[pl.ds(b_id * BROWS, BROWS)])

    return body


def _decoder():
    K = 160
    NCHUNK = P // K

    @functools.partial(
        pl.kernel,
        out_type=jax.ShapeDtypeStruct((P * 4,), jnp.float32),
        mesh=plsc.VectorSubcoreMesh(**_MESH),
        compiler_params=_SC_PARAMS,
        scratch_types=[
            pltpu.VMEM((N * 4,), jnp.float32),
            pltpu.VMEM((N * 4,), jnp.float32),
            pltpu.VMEM((K,), jnp.int32),
            pltpu.VMEM((K,), jnp.int32),
            pltpu.VMEM((K * 4,), jnp.float32),
            pltpu.VMEM((L,), jnp.float32),
        ],
    )
    def body(ab4_hbm, srcp_hbm, dstp_hbm, bpad_hbm, out_hbm,
             a4, b4, srcb, dstb, outb, bvec):
        c = lax.axis_index("c")
        s = lax.axis_index("s")
        wid = s * NC + c
        pltpu.sync_copy(ab4_hbm.at[pl.ds(0, N * 4)], a4)
        pltpu.sync_copy(ab4_hbm.at[pl.ds(N * 4, N * 4)], b4)
        pltpu.sync_copy(bpad_hbm, bvec)
        bv = bvec[...]
        iota = lax.iota(jnp.int32, L)
        rep = lax.shift_right_logical(iota, 2)
        col = lax.bitwise_and(iota, lax.broadcast(jnp.int32(3), (L,)))

        def ch(t, carry):
            chunk = wid + t * NW

            @pl.when(chunk < NCHUNK)
            def _():
                base = chunk * K
                pltpu.sync_copy(srcp_hbm.at[pl.ds(base, K)], srcb)
                pltpu.sync_copy(dstp_hbm.at[pl.ds(base, K)], dstb)

                def v_(v, carry2):
                    pidx = v * 4 + rep
                    sn = plsc.load_gather(srcb, [pidx])
                    dn = plsc.load_gather(dstb, [pidx])
                    va = plsc.load_gather(a4, [sn * 4 + col])
                    vb = plsc.load_gather(b4, [dn * 4 + col])
                    outb[pl.ds(v * L, L)] = va + vb + bv
                    return carry2
                lax.fori_loop(0, K * 4 // L, v_, 0)
                pltpu.sync_copy(outb, out_hbm.at[pl.ds(base * 4, K * 4)])
            return carry
        lax.fori_loop(0, (NCHUNK + NW - 1) // NW, ch, 0)

    return body


def _tc_z(x, w):
    BN = 2000

    def kern(x_ref, w_ref, z_ref):
        z_ref[0] = jnp.dot(x_ref[...], w_ref[0],
                           preferred_element_type=jnp.float32)

    z = pl.pallas_call(
        kern,
        grid=(R, N // BN),
        in_specs=[
            pl.BlockSpec((BN, D), lambda r, i: (i, 0)),
            pl.BlockSpec((1, D, D), lambda r, i: (r, 0, 0)),
        ],
        out_specs=pl.BlockSpec((1, BN, D), lambda r, i: (r, i, 0)),
        out_shape=jax.ShapeDtypeStruct((R, N, D), jnp.float32),
    )(x, w)
    return z.reshape(R * N, D)


def _tc_combine(part, x, root, b, relu):
    BN = 2048

    def kern(p_ref, x_ref, r_ref, b_ref, o_ref):
        o = (p_ref[...] + b_ref[...]
             + jnp.dot(x_ref[...], r_ref[...],
                       preferred_element_type=jnp.float32))
        if relu:
            o = jnp.maximum(o, 0.0)
        o_ref[...] = o

    return pl.pallas_call(
        kern,
        grid=(NP // BN,),
        in_specs=[
            pl.BlockSpec((BN, D), lambda i: (i, 0)),
            pl.BlockSpec((BN, D), lambda i: (i, 0)),
            pl.BlockSpec((D, D), lambda i: (0, 0)),
            pl.BlockSpec((1, D), lambda i: (0, 0)),
        ],
        out_specs=pl.BlockSpec((BN, D), lambda i: (i, 0)),
        out_shape=jax.ShapeDtypeStruct((NP, D), jnp.float32),
    )(part, x, root, b)


def _tc_decprep(h, wdec_pad):
    BN = 2000

    def kern(h_ref, w_ref, o_ref):
        o_ref[0] = jnp.dot(h_ref[...], w_ref[0],
                           preferred_element_type=jnp.float32)

    return pl.pallas_call(
        kern,
        grid=(2, N // BN),
        in_specs=[
            pl.BlockSpec((BN, D), lambda g, i: (i, 0)),
            pl.BlockSpec((1, D, 4), lambda g, i: (g, 0, 0)),
        ],
        out_specs=pl.BlockSpec((1, BN, 4), lambda g, i: (g, i, 0)),
        out_shape=jax.ShapeDtypeStruct((2, N, 4), jnp.float32),
    )(h, wdec_pad)


def kernel(x, edge_index, edge_type, pairs, w1, root1, b1, w2, root2, b2,
           W_dec, b_dec):
    src = edge_index[0].astype(jnp.int32)
    dst = edge_index[1].astype(jnp.int32)
    typ = edge_type.astype(jnp.int32)

    hist = _count()(dst, typ)
    cnt = _tc_reduce_hist(hist)
    recg, recl, reci, cnts = _bucketize()(src, typ, dst, cnt)
    agg = _aggregate()

    z1 = _tc_z(x, w1)
    p1 = agg(z1, recg, recl, reci, cnts)
    h1 = _tc_combine(p1, x, root1, b1.reshape(1, D), relu=True)[:N]

    z2 = _tc_z(h1, w2)
    p2 = agg(z2, recg, recl, reci, cnts)
    h2 = _tc_combine(p2, h1, root2, b2.reshape(1, D), relu=False)[:N]

    wdp = jnp.pad(W_dec.reshape(2, D, 3), ((0, 0), (0, 0), (0, 1)))
    ab4 = _tc_decprep(h2, wdp).reshape(2 * N * 4)
    bpad = jnp.tile(jnp.pad(b_dec, (0, 1)), 4)
    psrc = pairs[:, 0].astype(jnp.int32)
    pdst = pairs[:, 1].astype(jnp.int32)
    out4 = _decoder()(ab4, psrc, pdst, bpad)
    return out4.reshape(P, 4)[:, :3]


# speedup vs baseline: 3.7865x; 3.7865x over previous
import functools

import jax
import jax.numpy as jnp
from jax import lax
from jax.experimental import pallas as pl
from jax.experimental.pallas import tpu as pltpu
from jax.experimental.pallas import tpu_sc as plsc

N = 10000
E = 320000
P = 100000
R = 8
D = 128
NC = 2
NS = 16
L = 16
NW = NC * NS
BINS = 81920
NB = 32
BROWS = 320
NP = NB * BROWS
EPT = E // NW
RECB = EPT + 8 * NB - 16

_MESH = dict(core_axis_name="c", subcore_axis_name="s")
_SC_PARAMS = pltpu.CompilerParams(needs_layout_passes=False)


def _count():
    K = 80

    @functools.partial(
        pl.kernel,
        out_type=jax.ShapeDtypeStruct((NW * BINS,), jnp.float32),
        mesh=plsc.VectorSubcoreMesh(**_MESH),
        compiler_params=_SC_PARAMS,
        scratch_types=[
            pltpu.VMEM((BINS,), jnp.float32),
            pltpu.VMEM((K,), jnp.int32),
            pltpu.VMEM((K,), jnp.int32),
        ],
    )
    def body(dst_hbm, typ_hbm, hist_hbm, cnt, dstb, typb):
        c = lax.axis_index("c")
        s = lax.axis_index("s")
        wid = s * NC + c
        zeros16 = lax.broadcast(jnp.float32(0), (L,))
        ones16 = lax.broadcast(jnp.float32(1), (L,))

        def zero_body(i, carry):
            cnt[pl.ds(i * L, L)] = zeros16
            return carry
        lax.fori_loop(0, BINS // L, zero_body, 0)

        def p1(i, carry):
            eb = wid * EPT + i * K
            pltpu.sync_copy(dst_hbm.at[pl.ds(eb, K)], dstb)
            pltpu.sync_copy(typ_hbm.at[pl.ds(eb, K)], typb)

            def inner(v, carry2):
                idx = dstb[pl.ds(v * L, L)] * R + typb[pl.ds(v * L, L)]
                plsc.addupdate_scatter(cnt, [idx], ones16)
                return carry2
            lax.fori_loop(0, K // L, inner, 0)
            return carry
        lax.fori_loop(0, EPT // K, p1, 0)

        pltpu.sync_copy(cnt, hist_hbm.at[pl.ds(wid * BINS, BINS)])

    return body


def _tc_reduce_hist(hist):
    BB = 8

    def kern(h_ref, o_ref):
        o_ref[...] = jnp.sum(h_ref[...], axis=0)

    out = pl.pallas_call(
        kern,
        grid=(BINS // (BB * 128),),
        in_specs=[pl.BlockSpec((NW, BB, 128), lambda i: (0, i, 0))],
        out_specs=pl.BlockSpec((BB, 128), lambda i: (i, 0)),
        out_shape=jax.ShapeDtypeStruct((BINS // 128, 128), jnp.float32),
    )(hist.reshape(NW, BINS // 128, 128))
    return out.reshape(BINS)


def _bucketize():
    K = 80

    @functools.partial(
        pl.kernel,
        out_type=[
            jax.ShapeDtypeStruct((NW * RECB + K,), jnp.int32),
            jax.ShapeDtypeStruct((NW * RECB + K,), jnp.int32),
            jax.ShapeDtypeStruct((NW * RECB + K,), jnp.int32),
            jax.ShapeDtypeStruct((NW * NB,), jnp.int32),
        ],
        mesh=plsc.VectorSubcoreMesh(**_MESH),
        compiler_params=_SC_PARAMS,
        scratch_types=[
            pltpu.VMEM((BINS,), jnp.float32),
            pltpu.VMEM((RECB,), jnp.int32),
            pltpu.VMEM((RECB,), jnp.int32),
            pltpu.VMEM((RECB,), jnp.int32),
            pltpu.VMEM((NB + L,), jnp.int32),
            pltpu.VMEM((NB,), jnp.float32),
            pltpu.VMEM((K,), jnp.int32),
            pltpu.VMEM((K,), jnp.int32),
            pltpu.VMEM((K,), jnp.int32),
            pltpu.VMEM((K,), jnp.int32),
            pltpu.VMEM((K,), jnp.int32),
            pltpu.VMEM((K,), jnp.int32),
            pltpu.VMEM((K,), jnp.int32),
            pltpu.SMEM((NB,), jnp.int32),
        ],
    )
    def body(src_hbm, typ_hbm, dst_hbm, cnt_hbm,
             recg_hbm, recl_hbm, reci_hbm, cnts_hbm,
             cntb, gblk, lblk, iblk, bcnt, bcntf, srcb, typb, dstb, bktb,
             gidxb, lrowb, ivib, offs):
        c = lax.axis_index("c")
        s = lax.axis_index("s")
        wid = s * NC + c
        zeros16 = lax.broadcast(jnp.float32(0), (L,))
        ones16 = lax.broadcast(jnp.float32(1), (L,))

        pltpu.sync_copy(cnt_hbm, cntb)
        for v in range(NB // L):
            bcntf[pl.ds(v * L, L)] = zeros16

        def p1(i, carry):
            eb = wid * EPT + i * K
            pltpu.sync_copy(dst_hbm.at[pl.ds(eb, K)], dstb)

            def inner(v, carry2):
                bv = dstb[pl.ds(v * L, L)] // BROWS
                plsc.addupdate_scatter(bcntf, [bv], ones16)
                return carry2
            lax.fori_loop(0, K // L, inner, 0)
            return carry
        lax.fori_loop(0, EPT // K, p1, 0)

        for v in range(NB // L):
            bcnt[pl.ds(v * L, L)] = bcntf[pl.ds(v * L, L)].astype(jnp.int32)
        pltpu.sync_copy(bcnt.at[pl.ds(0, NB)],
                        cnts_hbm.at[pl.ds(wid * NB, NB)])

        def cum(b, off):
            offs[b] = off
            n = plsc.load_gather(bcnt, [lax.broadcast(b, (L,))])[0]
            return off + ((n + 7) // 8) * 8
        lax.fori_loop(0, NB, cum, 0)

        def p3(i, carry):
            eb = wid * EPT + i * K
            pltpu.sync_copy(src_hbm.at[pl.ds(eb, K)], srcb)
            pltpu.sync_copy(typ_hbm.at[pl.ds(eb, K)], typb)
            pltpu.sync_copy(dst_hbm.at[pl.ds(eb, K)], dstb)

            def vect(v, carry2):
                sl = pl.ds(v * L, L)
                dv = dstb[sl]
                tv = typb[sl]
                bv = dv // BROWS
                bktb[sl] = bv
                lrowb[sl] = dv - bv * BROWS
                gidxb[sl] = tv * N + srcb[sl]
                cv = plsc.load_gather(cntb, [dv * R + tv])
                ivib[sl] = plsc.bitcast(1.0 / jnp.maximum(cv, 1.0), jnp.int32)
                return carry2
            lax.fori_loop(0, K // L, vect, 0)

            def place(b, carry2):
                def pv(v, off):
                    sl = pl.ds(v * L, L)
                    m = bktb[sl] == b
                    mi = m.astype(jnp.int32)
                    pos = off + plsc.cumsum(mi) - mi
                    plsc.store_scatter(gblk, [pos], gidxb[sl], mask=m)
                    plsc.store_scatter(lblk, [pos], lrowb[sl], mask=m)
                    plsc.store_scatter(iblk, [pos], ivib[sl], mask=m)
                    return off + jnp.sum(mi)
                offs[b] = lax.fori_loop(0, K // L, pv, offs[b])
                return carry2
            lax.fori_loop(0, NB, place, 0)
            return carry
        lax.fori_loop(0, EPT // K, p3, 0)

        pltpu.sync_copy(gblk, recg_hbm.at[pl.ds(wid * RECB, RECB)])
        pltpu.sync_copy(lblk, recl_hbm.at[pl.ds(wid * RECB, RECB)])
        pltpu.sync_copy(iblk, reci_hbm.at[pl.ds(wid * RECB, RECB)])

    return body


def _aggregate():
    K = 80

    @functools.partial(
        pl.kernel,
        out_type=jax.ShapeDtypeStruct((NP * D,), jnp.float32),
        mesh=plsc.VectorSubcoreMesh(**_MESH),
        compiler_params=_SC_PARAMS,
        scratch_types=[
            pltpu.VMEM((BROWS * D,), jnp.float32),
            pltpu.VMEM((NW * NB + L,), jnp.int32),
            pltpu.VMEM((K,), jnp.int32),
            pltpu.VMEM((K + L,), jnp.int32),
            pltpu.VMEM((K,), jnp.int32),
            pltpu.VMEM((K,), jnp.float32),
            pltpu.VMEM((K, D), jnp.float32),
            pltpu.SMEM((NW,), jnp.int32),
            pltpu.SemaphoreType.DMA,
            pltpu.SemaphoreType.DMA,
        ],
    )
    def body(z_hbm, recg_hbm, recl_hbm, reci_hbm, cnts_hbm, out_hbm,
             accf, cntsb, gidxb, lrowb, ivib, ivb, rows, offsB, gsem, rsem):
        c = lax.axis_index("c")
        s = lax.axis_index("s")
        wid = s * NC + c
        zeros16 = lax.broadcast(jnp.float32(0), (L,))
        iota = lax.iota(jnp.int32, L)

        def za(i, carry):
            accf[pl.ds(i * L, L)] = zeros16
            return carry
        lax.fori_loop(0, BROWS * D // L, za, 0)

        pltpu.sync_copy(cnts_hbm, cntsb.at[pl.ds(0, NW * NB)])

        def po(t, carry):
            def po2(bb, off):
                n = plsc.load_gather(
                    cntsb, [lax.broadcast(t * NB + bb, (L,))])[0]
                ne = ((n + 7) // 8) * 8
                return off + jnp.where(bb < wid, ne, 0)
            offsB[t] = lax.fori_loop(0, NB, po2, 0)
            return carry
        lax.fori_loop(0, NW, po, 0)

        def per_src(t, carry):
            n = plsc.load_gather(
                cntsb, [lax.broadcast(t * NB + wid, (L,))])[0]
            base = pl.multiple_of(t * RECB + offsB[t], 8)

            def chunk(i, carry2):
                cps = (
                    pltpu.make_async_copy(
                        recg_hbm.at[pl.ds(base + i * K, K)], gidxb, rsem),
                    pltpu.make_async_copy(
                        recl_hbm.at[pl.ds(base + i * K, K)],
                        lrowb.at[pl.ds(0, K)], rsem),
                    pltpu.make_async_copy(
                        reci_hbm.at[pl.ds(base + i * K, K)], ivib, rsem),
                )
                for cp in cps:
                    cp.start()
                for cp in cps:
                    cp.wait()
                nval = n - i * K

                def vect(v, carry3):
                    sl = pl.ds(v * L, L)
                    valid = (v * L + iota) < nval
                    gidxb[sl] = jnp.where(valid, gidxb[sl], 0)
                    lrowb[sl] = jnp.where(valid, lrowb[sl], 0)
                    iv = plsc.bitcast(ivib[sl], jnp.float32)
                    ivb[sl] = jnp.where(valid, iv, 0.0)
                    return carry3
                lax.fori_loop(0, K // L, vect, 0)

                pltpu.async_copy(z_hbm.at[gidxb], rows, gsem).wait()

                def upd(j, carry3):
                    iv = plsc.load_gather(ivb, [lax.broadcast(j, (L,))])
                    lbase = plsc.load_gather(
                        lrowb, [lax.broadcast(j, (L,))]) * D + iota
                    for q in range(D // L):
                        plsc.addupdate_scatter(
                            accf, [lbase + q * L],
                            rows[j, pl.ds(q * L, L)] * iv)
                    return carry3
                lax.fori_loop(0, K, upd, 0, unroll=2)
                return carry2
            lax.fori_loop(0, (n + K - 1) // K, chunk, 0)
            return carry
        lax.fori_loop(0, NW, per_src, 0)

        pltpu.sync_copy(
            accf, out_hbm.at[pl.ds(wid * BROWS * D, BROWS * D)])

    return body


def _decoder():
    K = 160
    NCHUNK = P // K

    @functools.partial(
        pl.kernel,
        out_type=jax.ShapeDtypeStruct((P * 4,), jnp.float32),
        mesh=plsc.VectorSubcoreMesh(**_MESH),
        compiler_params=_SC_PARAMS,
        scratch_types=[
            pltpu.VMEM((N * 4,), jnp.float32),
            pltpu.VMEM((N * 4,), jnp.float32),
            pltpu.VMEM((K,), jnp.int32),
            pltpu.VMEM((K,), jnp.int32),
            pltpu.VMEM((K * 4,), jnp.float32),
            pltpu.VMEM((L,), jnp.float32),
        ],
    )
    def body(ab4_hbm, srcp_hbm, dstp_hbm, bpad_hbm, out_hbm,
             a4, b4, srcb, dstb, outb, bvec):
        c = lax.axis_index("c")
        s = lax.axis_index("s")
        wid = s * NC + c
        pltpu.sync_copy(ab4_hbm.at[pl.ds(0, N * 4)], a4)
        pltpu.sync_copy(ab4_hbm.at[pl.ds(N * 4, N * 4)], b4)
        pltpu.sync_copy(bpad_hbm, bvec)
        bv = bvec[...]
        iota = lax.iota(jnp.int32, L)
        rep = lax.shift_right_logical(iota, 2)
        col = lax.bitwise_and(iota, lax.broadcast(jnp.int32(3), (L,)))

        def ch(t, carry):
            chunk = wid + t * NW

            @pl.when(chunk < NCHUNK)
            def _():
                base = chunk * K
                pltpu.sync_copy(srcp_hbm.at[pl.ds(base, K)], srcb)
                pltpu.sync_copy(dstp_hbm.at[pl.ds(base, K)], dstb)

                def v_(v, carry2):
                    pidx = v * 4 + rep
                    sn = plsc.load_gather(srcb, [pidx])
                    dn = plsc.load_gather(dstb, [pidx])
                    va = plsc.load_gather(a4, [sn * 4 + col])
                    vb = plsc.load_gather(b4, [dn * 4 + col])
                    outb[pl.ds(v * L, L)] = va + vb + bv
                    return carry2
                lax.fori_loop(0, K * 4 // L, v_, 0)
                pltpu.sync_copy(outb, out_hbm.at[pl.ds(base * 4, K * 4)])
            return carry
        lax.fori_loop(0, (NCHUNK + NW - 1) // NW, ch, 0)

    return body


def _tc_z(x, w):
    BN = 2000

    def kern(x_ref, w_ref, z_ref):
        z_ref[0] = jnp.dot(x_ref[...], w_ref[0],
                           preferred_element_type=jnp.float32)

    z = pl.pallas_call(
        kern,
        grid=(R, N // BN),
        in_specs=[
            pl.BlockSpec((BN, D), lambda r, i: (i, 0)),
            pl.BlockSpec((1, D, D), lambda r, i: (r, 0, 0)),
        ],
        out_specs=pl.BlockSpec((1, BN, D), lambda r, i: (r, i, 0)),
        out_shape=jax.ShapeDtypeStruct((R, N, D), jnp.float32),
    )(x, w)
    return z.reshape(R * N, D)


def _tc_combine(part, x, root, b, relu):
    BN = 2048

    def kern(p_ref, x_ref, r_ref, b_ref, o_ref):
        o = (p_ref[...] + b_ref[...]
             + jnp.dot(x_ref[...], r_ref[...],
                       preferred_element_type=jnp.float32))
        if relu:
            o = jnp.maximum(o, 0.0)
        o_ref[...] = o

    return pl.pallas_call(
        kern,
        grid=(NP // BN,),
        in_specs=[
            pl.BlockSpec((BN, D), lambda i: (i, 0)),
            pl.BlockSpec((BN, D), lambda i: (i, 0)),
            pl.BlockSpec((D, D), lambda i: (0, 0)),
            pl.BlockSpec((1, D), lambda i: (0, 0)),
        ],
        out_specs=pl.BlockSpec((BN, D), lambda i: (i, 0)),
        out_shape=jax.ShapeDtypeStruct((NP, D), jnp.float32),
    )(part, x, root, b)


def _tc_decprep(h, wdec_pad):
    BN = 2000

    def kern(h_ref, w_ref, o_ref):
        o_ref[0] = jnp.dot(h_ref[...], w_ref[0],
                           preferred_element_type=jnp.float32)

    return pl.pallas_call(
        kern,
        grid=(2, N // BN),
        in_specs=[
            pl.BlockSpec((BN, D), lambda g, i: (i, 0)),
            pl.BlockSpec((1, D, 4), lambda g, i: (g, 0, 0)),
        ],
        out_specs=pl.BlockSpec((1, BN, 4), lambda g, i: (g, i, 0)),
        out_shape=jax.ShapeDtypeStruct((2, N, 4), jnp.float32),
    )(h, wdec_pad)


def kernel(x, edge_index, edge_type, pairs, w1, root1, b1, w2, root2, b2,
           W_dec, b_dec):
    src = edge_index[0].astype(jnp.int32)
    dst = edge_index[1].astype(jnp.int32)
    typ = edge_type.astype(jnp.int32)

    hist = _count()(dst, typ)
    cnt = _tc_reduce_hist(hist)
    recg, recl, reci, cnts = _bucketize()(src, typ, dst, cnt)
    agg = _aggregate()

    z1 = _tc_z(x, w1)
    p1 = agg(z1, recg, recl, reci, cnts).reshape(NP, D)
    h1 = _tc_combine(p1, x, root1, b1.reshape(1, D), relu=True)[:N]

    z2 = _tc_z(h1, w2)
    p2 = agg(z2, recg, recl, reci, cnts).reshape(NP, D)
    h2 = _tc_combine(p2, h1, root2, b2.reshape(1, D), relu=False)[:N]

    wdp = jnp.pad(W_dec.reshape(2, D, 3), ((0, 0), (0, 0), (0, 1)))
    ab4 = _tc_decprep(h2, wdp).reshape(2 * N * 4)
    bpad = jnp.tile(jnp.pad(b_dec, (0, 1)), 4)
    psrc = pairs[:, 0].astype(jnp.int32)
    pdst = pairs[:, 1].astype(jnp.int32)
    out4 = _decoder()(ab4, psrc, pdst, bpad)
    return out4.reshape(P, 4)[:, :3]


# speedup vs baseline: 3.8082x; 1.0057x over previous
import functools

import jax
import jax.numpy as jnp
from jax import lax
from jax.experimental import pallas as pl
from jax.experimental.pallas import tpu as pltpu
from jax.experimental.pallas import tpu_sc as plsc

N = 10000
E = 320000
P = 100000
R = 8
D = 128
NC = 2
NS = 16
L = 16
NW = NC * NS
BINS = 81920
NB = 32
BROWS = 320
NP = NB * BROWS
EPT = E // NW
RECB = EPT + 8 * NB - 16

_MESH = dict(core_axis_name="c", subcore_axis_name="s")
_SC_PARAMS = pltpu.CompilerParams(needs_layout_passes=False)


def _count():
    K = 400

    @functools.partial(
        pl.kernel,
        out_type=jax.ShapeDtypeStruct((NW * BINS,), jnp.float32),
        mesh=plsc.VectorSubcoreMesh(**_MESH),
        compiler_params=_SC_PARAMS,
        scratch_types=[
            pltpu.VMEM((BINS,), jnp.float32),
            pltpu.VMEM((K,), jnp.int32),
            pltpu.VMEM((K,), jnp.int32),
            pltpu.SemaphoreType.DMA,
        ],
    )
    def body(dst_hbm, typ_hbm, hist_hbm, cnt, dstb, typb, rsem):
        c = lax.axis_index("c")
        s = lax.axis_index("s")
        wid = s * NC + c
        zeros16 = lax.broadcast(jnp.float32(0), (L,))
        ones16 = lax.broadcast(jnp.float32(1), (L,))

        def zero_body(i, carry):
            cnt[pl.ds(i * L, L)] = zeros16
            return carry
        lax.fori_loop(0, BINS // L, zero_body, 0)

        def p1(i, carry):
            eb = wid * EPT + i * K
            cps = (pltpu.make_async_copy(dst_hbm.at[pl.ds(eb, K)], dstb, rsem),
                   pltpu.make_async_copy(typ_hbm.at[pl.ds(eb, K)], typb, rsem))
            for cp in cps:
                cp.start()
            for cp in cps:
                cp.wait()

            def inner(v, carry2):
                idx = dstb[pl.ds(v * L, L)] * R + typb[pl.ds(v * L, L)]
                plsc.addupdate_scatter(cnt, [idx], ones16)
                return carry2
            lax.fori_loop(0, K // L, inner, 0)
            return carry
        lax.fori_loop(0, EPT // K, p1, 0)

        pltpu.sync_copy(cnt, hist_hbm.at[pl.ds(wid * BINS, BINS)])

    return body


def _tc_reduce_hist(hist):
    BB = 8

    def kern(h_ref, o_ref):
        o_ref[...] = jnp.sum(h_ref[...], axis=0)

    out = pl.pallas_call(
        kern,
        grid=(BINS // (BB * 128),),
        in_specs=[pl.BlockSpec((NW, BB, 128), lambda i: (0, i, 0))],
        out_specs=pl.BlockSpec((BB, 128), lambda i: (i, 0)),
        out_shape=jax.ShapeDtypeStruct((BINS // 128, 128), jnp.float32),
    )(hist.reshape(NW, BINS // 128, 128))
    return out.reshape(BINS)


def _bucketize():
    K = 400

    @functools.partial(
        pl.kernel,
        out_type=[
            jax.ShapeDtypeStruct((NW * RECB + K,), jnp.int32),
            jax.ShapeDtypeStruct((NW * RECB + K,), jnp.int32),
            jax.ShapeDtypeStruct((NW * RECB + K,), jnp.int32),
            jax.ShapeDtypeStruct((NW * NB,), jnp.int32),
        ],
        mesh=plsc.VectorSubcoreMesh(**_MESH),
        compiler_params=_SC_PARAMS,
        scratch_types=[
            pltpu.VMEM((BINS,), jnp.float32),
            pltpu.VMEM((RECB,), jnp.int32),
            pltpu.VMEM((RECB,), jnp.int32),
            pltpu.VMEM((RECB,), jnp.int32),
            pltpu.VMEM((NB + L,), jnp.int32),
            pltpu.VMEM((NB,), jnp.float32),
            pltpu.VMEM((K,), jnp.int32),
            pltpu.VMEM((K,), jnp.int32),
            pltpu.VMEM((K,), jnp.int32),
            pltpu.VMEM((K,), jnp.int32),
            pltpu.VMEM((K,), jnp.int32),
            pltpu.VMEM((K,), jnp.int32),
            pltpu.VMEM((K,), jnp.int32),
            pltpu.SMEM((NB,), jnp.int32),
            pltpu.SemaphoreType.DMA,
        ],
    )
    def body(src_hbm, typ_hbm, dst_hbm, cnt_hbm,
             recg_hbm, recl_hbm, reci_hbm, cnts_hbm,
             cntb, gblk, lblk, iblk, bcnt, bcntf, srcb, typb, dstb, bktb,
             gidxb, lrowb, ivib, offs, rsem):
        c = lax.axis_index("c")
        s = lax.axis_index("s")
        wid = s * NC + c
        zeros16 = lax.broadcast(jnp.float32(0), (L,))
        ones16 = lax.broadcast(jnp.float32(1), (L,))

        pltpu.sync_copy(cnt_hbm, cntb)
        for v in range(NB // L):
            bcntf[pl.ds(v * L, L)] = zeros16

        def p1(i, carry):
            eb = wid * EPT + i * K
            pltpu.sync_copy(dst_hbm.at[pl.ds(eb, K)], dstb)

            def inner(v, carry2):
                bv = dstb[pl.ds(v * L, L)] // BROWS
                plsc.addupdate_scatter(bcntf, [bv], ones16)
                return carry2
            lax.fori_loop(0, K // L, inner, 0)
            return carry
        lax.fori_loop(0, EPT // K, p1, 0)

        for v in range(NB // L):
            bcnt[pl.ds(v * L, L)] = bcntf[pl.ds(v * L, L)].astype(jnp.int32)
        pltpu.sync_copy(bcnt.at[pl.ds(0, NB)],
                        cnts_hbm.at[pl.ds(wid * NB, NB)])

        def cum(b, off):
            offs[b] = off
            n = plsc.load_gather(bcnt, [lax.broadcast(b, (L,))])[0]
            return off + ((n + 7) // 8) * 8
        lax.fori_loop(0, NB, cum, 0)

        def p3(i, carry):
            eb = wid * EPT + i * K
            cps = (pltpu.make_async_copy(src_hbm.at[pl.ds(eb, K)], srcb, rsem),
                   pltpu.make_async_copy(typ_hbm.at[pl.ds(eb, K)], typb, rsem),
                   pltpu.make_async_copy(dst_hbm.at[pl.ds(eb, K)], dstb, rsem))
            for cp in cps:
                cp.start()
            for cp in cps:
                cp.wait()

            def vect(v, carry2):
                sl = pl.ds(v * L, L)
                dv = dstb[sl]
                tv = typb[sl]
                bv = dv // BROWS
                bktb[sl] = bv
                lrowb[sl] = dv - bv * BROWS
                gidxb[sl] = tv * N + srcb[sl]
                cv = plsc.load_gather(cntb, [dv * R + tv])
                ivib[sl] = plsc.bitcast(1.0 / jnp.maximum(cv, 1.0), jnp.int32)
                return carry2
            lax.fori_loop(0, K // L, vect, 0)

            def place(b, carry2):
                def pv(v, off):
                    sl = pl.ds(v * L, L)
                    m = bktb[sl] == b
                    mi = m.astype(jnp.int32)
                    nm = jnp.sum(mi)

                    @pl.when(nm > 0)
                    def _():
                        pos = off + plsc.cumsum(mi) - mi
                        plsc.store_scatter(gblk, [pos], gidxb[sl], mask=m)
                        plsc.store_scatter(lblk, [pos], lrowb[sl], mask=m)
                        plsc.store_scatter(iblk, [pos], ivib[sl], mask=m)
                    return off + nm
                offs[b] = lax.fori_loop(0, K // L, pv, offs[b])
                return carry2
            lax.fori_loop(0, NB, place, 0)
            return carry
        lax.fori_loop(0, EPT // K, p3, 0)

        pltpu.sync_copy(gblk, recg_hbm.at[pl.ds(wid * RECB, RECB)])
        pltpu.sync_copy(lblk, recl_hbm.at[pl.ds(wid * RECB, RECB)])
        pltpu.sync_copy(iblk, reci_hbm.at[pl.ds(wid * RECB, RECB)])

    return body


def _aggregate():
    K = 80

    @functools.partial(
        pl.kernel,
        out_type=jax.ShapeDtypeStruct((NP * D,), jnp.float32),
        mesh=plsc.VectorSubcoreMesh(**_MESH),
        compiler_params=_SC_PARAMS,
        scratch_types=[
            pltpu.VMEM((BROWS * D,), jnp.float32),
            pltpu.VMEM((NW * NB + L,), jnp.int32),
            pltpu.VMEM((K,), jnp.int32),
            pltpu.VMEM((K + L,), jnp.int32),
            pltpu.VMEM((K,), jnp.int32),
            pltpu.VMEM((K,), jnp.float32),
            pltpu.VMEM((K, D), jnp.float32),
            pltpu.SMEM((NW,), jnp.int32),
            pltpu.SemaphoreType.DMA,
            pltpu.SemaphoreType.DMA,
        ],
    )
    def body(z_hbm, recg_hbm, recl_hbm, reci_hbm, cnts_hbm, out_hbm,
             accf, cntsb, gidxb, lrowb, ivib, ivb, rows, offsB, gsem, rsem):
        c = lax.axis_index("c")
        s = lax.axis_index("s")
        wid = s * NC + c
        zeros16 = lax.broadcast(jnp.float32(0), (L,))
        iota = lax.iota(jnp.int32, L)

        def za(i, carry):
            accf[pl.ds(i * L, L)] = zeros16
            return carry
        lax.fori_loop(0, BROWS * D // L, za, 0)

        pltpu.sync_copy(cnts_hbm, cntsb.at[pl.ds(0, NW * NB)])

        def po(t, carry):
            def po2(bb, off):
                n = plsc.load_gather(
                    cntsb, [lax.broadcast(t * NB + bb, (L,))])[0]
                ne = ((n + 7) // 8) * 8
                return off + jnp.where(bb < wid, ne, 0)
            offsB[t] = lax.fori_loop(0, NB, po2, 0)
            return carry
        lax.fori_loop(0, NW, po, 0)

        def per_src(t, carry):
            n = plsc.load_gather(
                cntsb, [lax.broadcast(t * NB + wid, (L,))])[0]
            base = pl.multiple_of(t * RECB + offsB[t], 8)

            def chunk(i, carry2):
                cps = (
                    pltpu.make_async_copy(
                        recg_hbm.at[pl.ds(base + i * K, K)], gidxb, rsem),
                    pltpu.make_async_copy(
                        recl_hbm.at[pl.ds(base + i * K, K)],
                        lrowb.at[pl.ds(0, K)], rsem),
                    pltpu.make_async_copy(
                        reci_hbm.at[pl.ds(base + i * K, K)], ivib, rsem),
                )
                for cp in cps:
                    cp.start()
                for cp in cps:
                    cp.wait()
                nval = n - i * K

                def vect(v, carry3):
                    sl = pl.ds(v * L, L)
                    valid = (v * L + iota) < nval
                    gidxb[sl] = jnp.where(valid, gidxb[sl], 0)
                    lrowb[sl] = jnp.where(valid, lrowb[sl], 0)
                    iv = plsc.bitcast(ivib[sl], jnp.float32)
                    ivb[sl] = jnp.where(valid, iv, 0.0)
                    return carry3
                lax.fori_loop(0, K // L, vect, 0)

                pltpu.async_copy(z_hbm.at[gidxb], rows, gsem).wait()

                def upd(j, carry3):
                    iv = plsc.load_gather(ivb, [lax.broadcast(j, (L,))])
                    lbase = plsc.load_gather(
                        lrowb, [lax.broadcast(j, (L,))]) * D + iota
                    for q in range(D // L):
                        plsc.addupdate_scatter(
                            accf, [lbase + q * L],
                            rows[j, pl.ds(q * L, L)] * iv)
                    return carry3
                lax.fori_loop(0, K, upd, 0, unroll=2)
                return carry2
            lax.fori_loop(0, (n + K - 1) // K, chunk, 0)
            return carry
        lax.fori_loop(0, NW, per_src, 0)

        pltpu.sync_copy(
            accf, out_hbm.at[pl.ds(wid * BROWS * D, BROWS * D)])

    return body


def _decoder():
    K = 160
    NCHUNK = P // K

    @functools.partial(
        pl.kernel,
        out_type=jax.ShapeDtypeStruct((P * 4,), jnp.float32),
        mesh=plsc.VectorSubcoreMesh(**_MESH),
        compiler_params=_SC_PARAMS,
        scratch_types=[
            pltpu.VMEM((N * 4,), jnp.float32),
            pltpu.VMEM((N * 4,), jnp.float32),
            pltpu.VMEM((K,), jnp.int32),
            pltpu.VMEM((K,), jnp.int32),
            pltpu.VMEM((K * 4,), jnp.float32),
            pltpu.VMEM((L,), jnp.float32),
        ],
    )
    def body(ab4_hbm, srcp_hbm, dstp_hbm, bpad_hbm, out_hbm,
             a4, b4, srcb, dstb, outb, bvec):
        c = lax.axis_index("c")
        s = lax.axis_index("s")
        wid = s * NC + c
        pltpu.sync_copy(ab4_hbm.at[pl.ds(0, N * 4)], a4)
        pltpu.sync_copy(ab4_hbm.at[pl.ds(N * 4, N * 4)], b4)
        pltpu.sync_copy(bpad_hbm, bvec)
        bv = bvec[...]
        iota = lax.iota(jnp.int32, L)
        rep = lax.shift_right_logical(iota, 2)
        col = lax.bitwise_and(iota, lax.broadcast(jnp.int32(3), (L,)))

        def ch(t, carry):
            chunk = wid + t * NW

            @pl.when(chunk < NCHUNK)
            def _():
                base = chunk * K
                pltpu.sync_copy(srcp_hbm.at[pl.ds(base, K)], srcb)
                pltpu.sync_copy(dstp_hbm.at[pl.ds(base, K)], dstb)

                def v_(v, carry2):
                    pidx = v * 4 + rep
                    sn = plsc.load_gather(srcb, [pidx])
                    dn = plsc.load_gather(dstb, [pidx])
                    va = plsc.load_gather(a4, [sn * 4 + col])
                    vb = plsc.load_gather(b4, [dn * 4 + col])
                    outb[pl.ds(v * L, L)] = va + vb + bv
                    return carry2
                lax.fori_loop(0, K * 4 // L, v_, 0)
                pltpu.sync_copy(outb, out_hbm.at[pl.ds(base * 4, K * 4)])
            return carry
        lax.fori_loop(0, (NCHUNK + NW - 1) // NW, ch, 0)

    return body


def _tc_z(x, w):
    BN = 2000

    def kern(x_ref, w_ref, z_ref):
        z_ref[0] = jnp.dot(x_ref[...], w_ref[0],
                           preferred_element_type=jnp.float32)

    z = pl.pallas_call(
        kern,
        grid=(R, N // BN),
        in_specs=[
            pl.BlockSpec((BN, D), lambda r, i: (i, 0)),
            pl.BlockSpec((1, D, D), lambda r, i: (r, 0, 0)),
        ],
        out_specs=pl.BlockSpec((1, BN, D), lambda r, i: (r, i, 0)),
        out_shape=jax.ShapeDtypeStruct((R, N, D), jnp.float32),
    )(x, w)
    return z.reshape(R * N, D)


def _tc_combine(part, x, root, b, relu):
    BN = 2048

    def kern(p_ref, x_ref, r_ref, b_ref, o_ref):
        o = (p_ref[...] + b_ref[...]
             + jnp.dot(x_ref[...], r_ref[...],
                       preferred_element_type=jnp.float32))
        if relu:
            o = jnp.maximum(o, 0.0)
        o_ref[...] = o

    return pl.pallas_call(
        kern,
        grid=(NP // BN,),
        in_specs=[
            pl.BlockSpec((BN, D), lambda i: (i, 0)),
            pl.BlockSpec((BN, D), lambda i: (i, 0)),
            pl.BlockSpec((D, D), lambda i: (0, 0)),
            pl.BlockSpec((1, D), lambda i: (0, 0)),
        ],
        out_specs=pl.BlockSpec((BN, D), lambda i: (i, 0)),
        out_shape=jax.ShapeDtypeStruct((NP, D), jnp.float32),
    )(part, x, root, b)


def _tc_decprep(h, wdec_pad):
    BN = 2000

    def kern(h_ref, w_ref, o_ref):
        o_ref[0] = jnp.dot(h_ref[...], w_ref[0],
                           preferred_element_type=jnp.float32)

    return pl.pallas_call(
        kern,
        grid=(2, N // BN),
        in_specs=[
            pl.BlockSpec((BN, D), lambda g, i: (i, 0)),
            pl.BlockSpec((1, D, 4), lambda g, i: (g, 0, 0)),
        ],
        out_specs=pl.BlockSpec((1, BN, 4), lambda g, i: (g, i, 0)),
        out_shape=jax.ShapeDtypeStruct((2, N, 4), jnp.float32),
    )(h, wdec_pad)


def kernel(x, edge_index, edge_type, pairs, w1, root1, b1, w2, root2, b2,
           W_dec, b_dec):
    src = edge_index[0].astype(jnp.int32)
    dst = edge_index[1].astype(jnp.int32)
    typ = edge_type.astype(jnp.int32)

    hist = _count()(dst, typ)
    cnt = _tc_reduce_hist(hist)
    recg, recl, reci, cnts = _bucketize()(src, typ, dst, cnt)
    agg = _aggregate()

    z1 = _tc_z(x, w1)
    p1 = agg(z1, recg, recl, reci, cnts).reshape(NP, D)
    h1 = _tc_combine(p1, x, root1, b1.reshape(1, D), relu=True)[:N]

    z2 = _tc_z(h1, w2)
    p2 = agg(z2, recg, recl, reci, cnts).reshape(NP, D)
    h2 = _tc_combine(p2, h1, root2, b2.reshape(1, D), relu=False)[:N]

    wdp = jnp.pad(W_dec.reshape(2, D, 3), ((0, 0), (0, 0), (0, 1)))
    ab4 = _tc_decprep(h2, wdp).reshape(2 * N * 4)
    bpad = jnp.tile(jnp.pad(b_dec, (0, 1)), 4)
    psrc = pairs[:, 0].astype(jnp.int32)
    pdst = pairs[:, 1].astype(jnp.int32)
    out4 = _decoder()(ab4, psrc, pdst, bpad)
    return out4.reshape(P, 4)[:, :3]


# speedup vs baseline: 3.8112x; 1.0008x over previous
import functools

import jax
import jax.numpy as jnp
from jax import lax
from jax.experimental import pallas as pl
from jax.experimental.pallas import tpu as pltpu
from jax.experimental.pallas import tpu_sc as plsc

N = 10000
E = 320000
P = 100000
R = 8
D = 128
NC = 2
NS = 16
L = 16
NW = NC * NS
BINS = 81920
NB = 32
BROWS = 320
NP = NB * BROWS
EPT = E // NW
RECB = EPT + 8 * NB - 16

_MESH = dict(core_axis_name="c", subcore_axis_name="s")
_SC_PARAMS = pltpu.CompilerParams(needs_layout_passes=False)


def _count():
    K = 400

    @functools.partial(
        pl.kernel,
        out_type=jax.ShapeDtypeStruct((NW * BINS,), jnp.float32),
        mesh=plsc.VectorSubcoreMesh(**_MESH),
        compiler_params=_SC_PARAMS,
        scratch_types=[
            pltpu.VMEM((BINS,), jnp.float32),
            pltpu.VMEM((K,), jnp.int32),
            pltpu.VMEM((K,), jnp.int32),
            pltpu.SemaphoreType.DMA,
        ],
    )
    def body(dst_hbm, typ_hbm, hist_hbm, cnt, dstb, typb, rsem):
        c = lax.axis_index("c")
        s = lax.axis_index("s")
        wid = s * NC + c
        zeros16 = lax.broadcast(jnp.float32(0), (L,))
        ones16 = lax.broadcast(jnp.float32(1), (L,))

        def zero_body(i, carry):
            cnt[pl.ds(i * L, L)] = zeros16
            return carry
        lax.fori_loop(0, BINS // L, zero_body, 0)

        def p1(i, carry):
            eb = wid * EPT + i * K
            cps = (pltpu.make_async_copy(dst_hbm.at[pl.ds(eb, K)], dstb, rsem),
                   pltpu.make_async_copy(typ_hbm.at[pl.ds(eb, K)], typb, rsem))
            for cp in cps:
                cp.start()
            for cp in cps:
                cp.wait()

            def inner(v, carry2):
                idx = dstb[pl.ds(v * L, L)] * R + typb[pl.ds(v * L, L)]
                plsc.addupdate_scatter(cnt, [idx], ones16)
                return carry2
            lax.fori_loop(0, K // L, inner, 0)
            return carry
        lax.fori_loop(0, EPT // K, p1, 0)

        pltpu.sync_copy(cnt, hist_hbm.at[pl.ds(wid * BINS, BINS)])

    return body


def _tc_reduce_hist(hist):
    BB = 8

    def kern(h_ref, o_ref):
        o_ref[...] = jnp.sum(h_ref[...], axis=0)

    out = pl.pallas_call(
        kern,
        grid=(BINS // (BB * 128),),
        in_specs=[pl.BlockSpec((NW, BB, 128), lambda i: (0, i, 0))],
        out_specs=pl.BlockSpec((BB, 128), lambda i: (i, 0)),
        out_shape=jax.ShapeDtypeStruct((BINS // 128, 128), jnp.float32),
    )(hist.reshape(NW, BINS // 128, 128))
    return out.reshape(BINS)


def _bucketize():
    K = 400

    @functools.partial(
        pl.kernel,
        out_type=[
            jax.ShapeDtypeStruct((NW * RECB + K,), jnp.int32),
            jax.ShapeDtypeStruct((NW * RECB + K,), jnp.int32),
            jax.ShapeDtypeStruct((NW * RECB + K,), jnp.int32),
            jax.ShapeDtypeStruct((NW * NB,), jnp.int32),
        ],
        mesh=plsc.VectorSubcoreMesh(**_MESH),
        compiler_params=_SC_PARAMS,
        scratch_types=[
            pltpu.VMEM((BINS,), jnp.float32),
            pltpu.VMEM((RECB,), jnp.int32),
            pltpu.VMEM((RECB,), jnp.int32),
            pltpu.VMEM((RECB,), jnp.int32),
            pltpu.VMEM((NB + L,), jnp.int32),
            pltpu.VMEM((NB,), jnp.float32),
            pltpu.VMEM((K,), jnp.int32),
            pltpu.VMEM((K,), jnp.int32),
            pltpu.VMEM((K,), jnp.int32),
            pltpu.VMEM((K,), jnp.int32),
            pltpu.VMEM((K,), jnp.int32),
            pltpu.VMEM((K,), jnp.int32),
            pltpu.VMEM((K,), jnp.int32),
            pltpu.SMEM((NB,), jnp.int32),
            pltpu.SemaphoreType.DMA,
        ],
    )
    def body(src_hbm, typ_hbm, dst_hbm, cnt_hbm,
             recg_hbm, recl_hbm, reci_hbm, cnts_hbm,
             cntb, gblk, lblk, iblk, bcnt, bcntf, srcb, typb, dstb, bktb,
             gidxb, lrowb, ivib, offs, rsem):
        c = lax.axis_index("c")
        s = lax.axis_index("s")
        wid = s * NC + c
        zeros16 = lax.broadcast(jnp.float32(0), (L,))
        ones16 = lax.broadcast(jnp.float32(1), (L,))

        pltpu.sync_copy(cnt_hbm, cntb)
        for v in range(NB // L):
            bcntf[pl.ds(v * L, L)] = zeros16

        def p1(i, carry):
            eb = wid * EPT + i * K
            pltpu.sync_copy(dst_hbm.at[pl.ds(eb, K)], dstb)

            def inner(v, carry2):
                bv = dstb[pl.ds(v * L, L)] // BROWS
                plsc.addupdate_scatter(bcntf, [bv], ones16)
                return carry2
            lax.fori_loop(0, K // L, inner, 0)
            return carry
        lax.fori_loop(0, EPT // K, p1, 0)

        for v in range(NB // L):
            bcnt[pl.ds(v * L, L)] = bcntf[pl.ds(v * L, L)].astype(jnp.int32)
        pltpu.sync_copy(bcnt.at[pl.ds(0, NB)],
                        cnts_hbm.at[pl.ds(wid * NB, NB)])

        def cum(b, off):
            offs[b] = off
            n = plsc.load_gather(bcnt, [lax.broadcast(b, (L,))])[0]
            return off + ((n + 7) // 8) * 8
        lax.fori_loop(0, NB, cum, 0)

        def p3(i, carry):
            eb = wid * EPT + i * K
            cps = (pltpu.make_async_copy(src_hbm.at[pl.ds(eb, K)], srcb, rsem),
                   pltpu.make_async_copy(typ_hbm.at[pl.ds(eb, K)], typb, rsem),
                   pltpu.make_async_copy(dst_hbm.at[pl.ds(eb, K)], dstb, rsem))
            for cp in cps:
                cp.start()
            for cp in cps:
                cp.wait()

            def vect(v, carry2):
                sl = pl.ds(v * L, L)
                dv = dstb[sl]
                tv = typb[sl]
                bv = dv // BROWS
                bktb[sl] = bv
                lrowb[sl] = dv - bv * BROWS
                gidxb[sl] = tv * N + srcb[sl]
                cv = plsc.load_gather(cntb, [dv * R + tv])
                ivib[sl] = plsc.bitcast(1.0 / jnp.maximum(cv, 1.0), jnp.int32)
                return carry2
            lax.fori_loop(0, K // L, vect, 0)

            def place(b, carry2):
                def pv(v, off):
                    sl = pl.ds(v * L, L)
                    m = bktb[sl] == b
                    mi = m.astype(jnp.int32)
                    nm = jnp.sum(mi)

                    @pl.when(nm > 0)
                    def _():
                        pos = off + plsc.cumsum(mi) - mi
                        plsc.store_scatter(gblk, [pos], gidxb[sl], mask=m)
                        plsc.store_scatter(lblk, [pos], lrowb[sl], mask=m)
                        plsc.store_scatter(iblk, [pos], ivib[sl], mask=m)
                    return off + nm
                offs[b] = lax.fori_loop(0, K // L, pv, offs[b])
                return carry2
            lax.fori_loop(0, NB, place, 0)
            return carry
        lax.fori_loop(0, EPT // K, p3, 0)

        pltpu.sync_copy(gblk, recg_hbm.at[pl.ds(wid * RECB, RECB)])
        pltpu.sync_copy(lblk, recl_hbm.at[pl.ds(wid * RECB, RECB)])
        pltpu.sync_copy(iblk, reci_hbm.at[pl.ds(wid * RECB, RECB)])

    return body


def _aggregate():
    K = 80
    GSUB = 5

    @functools.partial(
        pl.kernel,
        out_type=jax.ShapeDtypeStruct((NP * D,), jnp.float32),
        mesh=plsc.VectorSubcoreMesh(**_MESH),
        compiler_params=_SC_PARAMS,
        scratch_types=[
            pltpu.VMEM((BROWS * D,), jnp.float32),
            pltpu.VMEM((NW * NB + L,), jnp.int32),
            pltpu.VMEM((K,), jnp.int32),
            pltpu.VMEM((K + L,), jnp.int32),
            pltpu.VMEM((K,), jnp.int32),
            pltpu.VMEM((K,), jnp.float32),
            pltpu.VMEM((K, D), jnp.float32),
            pltpu.SMEM((NW,), jnp.int32),
            pltpu.SemaphoreType.DMA,
            pltpu.SemaphoreType.DMA,
        ],
    )
    def body(z_hbm, recg_hbm, recl_hbm, reci_hbm, cnts_hbm, out_hbm,
             accf, cntsb, gidxb, lrowb, ivib, ivb, rows, offsB, gsem, rsem):
        c = lax.axis_index("c")
        s = lax.axis_index("s")
        wid = s * NC + c
        zeros16 = lax.broadcast(jnp.float32(0), (L,))
        iota = lax.iota(jnp.int32, L)

        def za(i, carry):
            accf[pl.ds(i * L, L)] = zeros16
            return carry
        lax.fori_loop(0, BROWS * D // L, za, 0)

        pltpu.sync_copy(cnts_hbm, cntsb.at[pl.ds(0, NW * NB)])

        def po(t, carry):
            def po2(bb, off):
                n = plsc.load_gather(
                    cntsb, [lax.broadcast(t * NB + bb, (L,))])[0]
                ne = ((n + 7) // 8) * 8
                return off + jnp.where(bb < wid, ne, 0)
            offsB[t] = lax.fori_loop(0, NB, po2, 0)
            return carry
        lax.fori_loop(0, NW, po, 0)

        def per_src(t, carry):
            n = plsc.load_gather(
                cntsb, [lax.broadcast(t * NB + wid, (L,))])[0]
            base = pl.multiple_of(t * RECB + offsB[t], 8)

            def chunk(i, carry2):
                cps = (
                    pltpu.make_async_copy(
                        recg_hbm.at[pl.ds(base + i * K, K)], gidxb, rsem),
                    pltpu.make_async_copy(
                        recl_hbm.at[pl.ds(base + i * K, K)],
                        lrowb.at[pl.ds(0, K)], rsem),
                    pltpu.make_async_copy(
                        reci_hbm.at[pl.ds(base + i * K, K)], ivib, rsem),
                )
                for cp in cps:
                    cp.start()
                for cp in cps:
                    cp.wait()
                nval = n - i * K

                def vect(v, carry3):
                    sl = pl.ds(v * L, L)
                    valid = (v * L + iota) < nval
                    gidxb[sl] = jnp.where(valid, gidxb[sl], 0)
                    lrowb[sl] = jnp.where(valid, lrowb[sl], 0)
                    iv = plsc.bitcast(ivib[sl], jnp.float32)
                    ivb[sl] = jnp.where(valid, iv, 0.0)
                    return carry3
                lax.fori_loop(0, K // L, vect, 0)

                gcps = tuple(
                    pltpu.make_async_copy(
                        z_hbm.at[gidxb.at[pl.ds(gsub * (K // GSUB), K // GSUB)]],
                        rows.at[pl.ds(gsub * (K // GSUB), K // GSUB)],
                        gsem)
                    for gsub in range(GSUB))
                for cp in gcps:
                    cp.start()
                for cp in gcps:
                    cp.wait()

                def upd(j, carry3):
                    iv = plsc.load_gather(ivb, [lax.broadcast(j, (L,))])
                    lbase = plsc.load_gather(
                        lrowb, [lax.broadcast(j, (L,))]) * D + iota
                    for q in range(D // L):
                        plsc.addupdate_scatter(
                            accf, [lbase + q * L],
                            rows[j, pl.ds(q * L, L)] * iv)
                    return carry3
                lax.fori_loop(0, K, upd, 0, unroll=2)
                return carry2
            lax.fori_loop(0, (n + K - 1) // K, chunk, 0)
            return carry
        lax.fori_loop(0, NW, per_src, 0)

        pltpu.sync_copy(
            accf, out_hbm.at[pl.ds(wid * BROWS * D, BROWS * D)])

    return body


def _decoder():
    K = 160
    NCHUNK = P // K

    @functools.partial(
        pl.kernel,
        out_type=jax.ShapeDtypeStruct((P * 4,), jnp.float32),
        mesh=plsc.VectorSubcoreMesh(**_MESH),
        compiler_params=_SC_PARAMS,
        scratch_types=[
            pltpu.VMEM((N * 4,), jnp.float32),
            pltpu.VMEM((N * 4,), jnp.float32),
            pltpu.VMEM((K,), jnp.int32),
            pltpu.VMEM((K,), jnp.int32),
            pltpu.VMEM((K * 4,), jnp.float32),
            pltpu.VMEM((L,), jnp.float32),
        ],
    )
    def body(ab4_hbm, srcp_hbm, dstp_hbm, bpad_hbm, out_hbm,
             a4, b4, srcb, dstb, outb, bvec):
        c = lax.axis_index("c")
        s = lax.axis_index("s")
        wid = s * NC + c
        pltpu.sync_copy(ab4_hbm.at[pl.ds(0, N * 4)], a4)
        pltpu.sync_copy(ab4_hbm.at[pl.ds(N * 4, N * 4)], b4)
        pltpu.sync_copy(bpad_hbm, bvec)
        bv = bvec[...]
        iota = lax.iota(jnp.int32, L)
        rep = lax.shift_right_logical(iota, 2)
        col = lax.bitwise_and(iota, lax.broadcast(jnp.int32(3), (L,)))

        def ch(t, carry):
            chunk = wid + t * NW

            @pl.when(chunk < NCHUNK)
            def _():
                base = chunk * K
                pltpu.sync_copy(srcp_hbm.at[pl.ds(base, K)], srcb)
                pltpu.sync_copy(dstp_hbm.at[pl.ds(base, K)], dstb)

                def v_(v, carry2):
                    pidx = v * 4 + rep
                    sn = plsc.load_gather(srcb, [pidx])
                    dn = plsc.load_gather(dstb, [pidx])
                    va = plsc.load_gather(a4, [sn * 4 + col])
                    vb = plsc.load_gather(b4, [dn * 4 + col])
                    outb[pl.ds(v * L, L)] = va + vb + bv
                    return carry2
                lax.fori_loop(0, K * 4 // L, v_, 0)
                pltpu.sync_copy(outb, out_hbm.at[pl.ds(base * 4, K * 4)])
            return carry
        lax.fori_loop(0, (NCHUNK + NW - 1) // NW, ch, 0)

    return body


def _tc_z(x, w):
    BN = 2000

    def kern(x_ref, w_ref, z_ref):
        z_ref[0] = jnp.dot(x_ref[...], w_ref[0],
                           preferred_element_type=jnp.float32)

    z = pl.pallas_call(
        kern,
        grid=(R, N // BN),
        in_specs=[
            pl.BlockSpec((BN, D), lambda r, i: (i, 0)),
            pl.BlockSpec((1, D, D), lambda r, i: (r, 0, 0)),
        ],
        out_specs=pl.BlockSpec((1, BN, D), lambda r, i: (r, i, 0)),
        out_shape=jax.ShapeDtypeStruct((R, N, D), jnp.float32),
    )(x, w)
    return z.reshape(R * N, D)


def _tc_combine(part, x, root, b, relu):
    BN = 2048

    def kern(p_ref, x_ref, r_ref, b_ref, o_ref):
        o = (p_ref[...] + b_ref[...]
             + jnp.dot(x_ref[...], r_ref[...],
                       preferred_element_type=jnp.float32))
        if relu:
            o = jnp.maximum(o, 0.0)
        o_ref[...] = o

    return pl.pallas_call(
        kern,
        grid=(NP // BN,),
        in_specs=[
            pl.BlockSpec((BN, D), lambda i: (i, 0)),
            pl.BlockSpec((BN, D), lambda i: (i, 0)),
            pl.BlockSpec((D, D), lambda i: (0, 0)),
            pl.BlockSpec((1, D), lambda i: (0, 0)),
        ],
        out_specs=pl.BlockSpec((BN, D), lambda i: (i, 0)),
        out_shape=jax.ShapeDtypeStruct((NP, D), jnp.float32),
    )(part, x, root, b)


def _tc_decprep(h, wdec_pad):
    BN = 2000

    def kern(h_ref, w_ref, o_ref):
        o_ref[0] = jnp.dot(h_ref[...], w_ref[0],
                           preferred_element_type=jnp.float32)

    return pl.pallas_call(
        kern,
        grid=(2, N // BN),
        in_specs=[
            pl.BlockSpec((BN, D), lambda g, i: (i, 0)),
            pl.BlockSpec((1, D, 4), lambda g, i: (g, 0, 0)),
        ],
        out_specs=pl.BlockSpec((1, BN, 4), lambda g, i: (g, i, 0)),
        out_shape=jax.ShapeDtypeStruct((2, N, 4), jnp.float32),
    )(h, wdec_pad)


def kernel(x, edge_index, edge_type, pairs, w1, root1, b1, w2, root2, b2,
           W_dec, b_dec):
    src = edge_index[0].astype(jnp.int32)
    dst = edge_index[1].astype(jnp.int32)
    typ = edge_type.astype(jnp.int32)

    hist = _count()(dst, typ)
    cnt = _tc_reduce_hist(hist)
    recg, recl, reci, cnts = _bucketize()(src, typ, dst, cnt)
    agg = _aggregate()

    z1 = _tc_z(x, w1)
    p1 = agg(z1, recg, recl, reci, cnts).reshape(NP, D)
    h1 = _tc_combine(p1, x, root1, b1.reshape(1, D), relu=True)[:N]

    z2 = _tc_z(h1, w2)
    p2 = agg(z2, recg, recl, reci, cnts).reshape(NP, D)
    h2 = _tc_combine(p2, h1, root2, b2.reshape(1, D), relu=False)[:N]

    wdp = jnp.pad(W_dec.reshape(2, D, 3), ((0, 0), (0, 0), (0, 1)))
    ab4 = _tc_decprep(h2, wdp).reshape(2 * N * 4)
    bpad = jnp.tile(jnp.pad(b_dec, (0, 1)), 4)
    psrc = pairs[:, 0].astype(jnp.int32)
    pdst = pairs[:, 1].astype(jnp.int32)
    out4 = _decoder()(ab4, psrc, pdst, bpad)
    return out4.reshape(P, 4)[:, :3]
